# baseline (speedup 1.0000x reference)
import functools

import jax
import jax.numpy as jnp
from jax import lax
from jax.experimental import pallas as pl
from jax.experimental.pallas import tpu as pltpu
from jax.experimental.pallas import tpu_sc as plsc

EMB_DIM = 128
BATCH = 16384
_LANES = 16
_CHUNKS = EMB_DIM // _LANES

_info = plsc.get_sparse_core_info()
_NC, _NS = _info.num_cores, _info.num_subcores
_NW = _NC * _NS
_BPW = BATCH // _NW
_NBUF = 4
_ROWS_PER_BUF = _BPW // _NBUF

_mesh = plsc.VectorSubcoreMesh(core_axis_name="c", subcore_axis_name="s")

_GATHER_DNUMS = lax.GatherDimensionNumbers(
    offset_dims=(), collapsed_slice_dims=(0,), start_index_map=(0,))


def _bcast_lane(vec, r):
    idx = jnp.full((_LANES, 1), r, jnp.int32)
    return lax.gather(vec, idx, _GATHER_DNUMS, (1,),
                      mode=lax.GatherScatterMode.PROMISE_IN_BOUNDS)


@functools.partial(
    pl.kernel,
    mesh=_mesh,
    out_type=jax.ShapeDtypeStruct((BATCH, EMB_DIM), jnp.float32),
    scratch_types=[
        pltpu.VMEM((_BPW,), jnp.int32),
        pltpu.VMEM((3, EMB_DIM), jnp.float32),
        pltpu.VMEM((_BPW, EMB_DIM), jnp.float32),
        pltpu.SemaphoreType.DMA,
    ],
)
def _lookup(idx_hbm, table_hbm, out_hbm, idx_v, table_v, rows_v, osem):
    wid = lax.axis_index("s") * _NC + lax.axis_index("c")
    base = wid * _BPW
    pltpu.sync_copy(table_hbm, table_v)
    pltpu.sync_copy(idx_hbm.at[pl.ds(base, _BPW)], idx_v)

    t = [[table_v[k, pl.ds(j * _LANES, _LANES)] for j in range(_CHUNKS)]
         for k in range(3)]
    d0 = [t[0][j] - t[2][j] for j in range(_CHUNKS)]
    d1 = [t[1][j] - t[2][j] for j in range(_CHUNKS)]

    one = jnp.ones((_LANES,), jnp.int32)
    out_copies = []
    groups_per_buf = _ROWS_PER_BUF // _LANES
    for buf in range(_NBUF):
        def group_body(g, carry):
            b0 = g * _LANES
            idx16 = idx_v[pl.ds(b0, _LANES)]
            for r in range(_LANES):
                bc = _bcast_lane(idx16, r)
                o = jnp.minimum(bc, one)
                w0 = (one - o).astype(jnp.float32)
                w1 = (o - jnp.maximum(bc - one, 0)).astype(jnp.float32)
                for j in range(_CHUNKS):
                    v = t[2][j] + w0 * d0[j] + w1 * d1[j]
                    rows_v[b0 + r, pl.ds(j * _LANES, _LANES)] = v
            return carry

        lax.fori_loop(buf * groups_per_buf, (buf + 1) * groups_per_buf,
                      group_body, 0)
        out_copies.append(pltpu.async_copy(
            rows_v.at[pl.ds(buf * _ROWS_PER_BUF, _ROWS_PER_BUF)],
            out_hbm.at[pl.ds(base + buf * _ROWS_PER_BUF, _ROWS_PER_BUF)],
            osem))
    for cp in out_copies:
        cp.wait()


def kernel(x, table):
    idx = jnp.reshape(x, (BATCH,)).astype(jnp.int32)
    return _lookup(idx, table)


# speedup vs baseline: 1.0154x; 1.0154x over previous
import functools

import jax
import jax.numpy as jnp
from jax import lax
from jax.experimental import pallas as pl
from jax.experimental.pallas import tpu as pltpu
from jax.experimental.pallas import tpu_sc as plsc

EMB_DIM = 128
BATCH = 16384
_LANES = 16
_REP = 256

_info = plsc.get_sparse_core_info()
_NC, _NS = _info.num_cores, _info.num_subcores
_NW = _NC * _NS
_BPW = BATCH // _NW
_NBUF = 4
_ROWS_PER_BUF = _BPW // _NBUF

_mesh = plsc.VectorSubcoreMesh(core_axis_name="c", subcore_axis_name="s")


@functools.partial(
    pl.kernel,
    mesh=_mesh,
    out_type=jax.ShapeDtypeStruct((BATCH, EMB_DIM), jnp.float32),
    scratch_types=[
        pltpu.VMEM((_BPW,), jnp.int32),
        pltpu.VMEM((_BPW, EMB_DIM), jnp.float32),
        pltpu.SemaphoreType.DMA,
        pltpu.SemaphoreType.DMA,
        pltpu.SemaphoreType.DMA,
        pltpu.SemaphoreType.DMA,
        pltpu.SemaphoreType.DMA,
        pltpu.SemaphoreType.DMA,
        pltpu.SemaphoreType.DMA,
        pltpu.SemaphoreType.DMA,
        pltpu.SemaphoreType.DMA,
    ],
)
def _lookup(idx_hbm, table_hbm, out_hbm, idx_v, rows_v,
            i0, i1, i2, i3, g0, g1, g2, g3, osem):
    isems = [i0, i1, i2, i3]
    gsems = [g0, g1, g2, g3]
    wid = lax.axis_index("s") * _NC + lax.axis_index("c")
    base = wid * _BPW

    idx_copies = [
        pltpu.async_copy(
            idx_hbm.at[pl.ds(base + buf * _ROWS_PER_BUF, _ROWS_PER_BUF)],
            idx_v.at[pl.ds(buf * _ROWS_PER_BUF, _ROWS_PER_BUF)],
            isems[buf])
        for buf in range(_NBUF)
    ]

    lane3 = lax.iota(jnp.int32, _LANES) * 3
    gather_copies = []
    for buf in range(_NBUF):
        idx_copies[buf].wait()
        for g in range(_ROWS_PER_BUF // _LANES):
            b0 = buf * _ROWS_PER_BUF + g * _LANES
            slot0 = ((base + b0) % _REP) * 3
            idx_v[pl.ds(b0, _LANES)] = (
                idx_v[pl.ds(b0, _LANES)] + (slot0 + lane3))
        rsl = pl.ds(buf * _ROWS_PER_BUF, _ROWS_PER_BUF)
        gather_copies.append(pltpu.async_copy(
            table_hbm.at[idx_v.at[rsl]], rows_v.at[rsl], gsems[buf]))

    out_copies = []
    for buf in range(_NBUF):
        gather_copies[buf].wait()
        rsl = pl.ds(buf * _ROWS_PER_BUF, _ROWS_PER_BUF)
        out_copies.append(pltpu.async_copy(
            rows_v.at[rsl],
            out_hbm.at[pl.ds(base + buf * _ROWS_PER_BUF, _ROWS_PER_BUF)],
            osem))
    for cp in out_copies:
        cp.wait()


def kernel(x, table):
    idx = jnp.reshape(x, (BATCH,)).astype(jnp.int32)
    table_rep = jnp.tile(table, (_REP, 1))
    return _lookup(idx, table_rep)


# speedup vs baseline: 1.0599x; 1.0438x over previous
import functools

import jax
import jax.numpy as jnp
from jax import lax
from jax.experimental import pallas as pl
from jax.experimental.pallas import tpu as pltpu
from jax.experimental.pallas import tpu_sc as plsc

EMB_DIM = 128
BATCH = 16384
_LANES = 16
_CHUNKS = EMB_DIM // _LANES
_REP = 256

_info = plsc.get_sparse_core_info()
_NC, _NS = _info.num_cores, _info.num_subcores
_NW = _NC * _NS
_BPW = BATCH // _NW
_NBUF = 4
_ROWS_PER_BUF = _BPW // _NBUF
_GPB = _ROWS_PER_BUF // _LANES
_GATHER_BUF = _NBUF - 1

_mesh = plsc.VectorSubcoreMesh(core_axis_name="c", subcore_axis_name="s")

_GATHER_DNUMS = lax.GatherDimensionNumbers(
    offset_dims=(), collapsed_slice_dims=(0,), start_index_map=(0,))


def _bcast_lane(vec, r):
    idx = jnp.full((_LANES, 1), r, jnp.int32)
    return lax.gather(vec, idx, _GATHER_DNUMS, (1,),
                      mode=lax.GatherScatterMode.PROMISE_IN_BOUNDS)


@functools.partial(
    pl.kernel,
    mesh=_mesh,
    out_type=jax.ShapeDtypeStruct((BATCH, EMB_DIM), jnp.float32),
    scratch_types=[
        pltpu.VMEM((_BPW,), jnp.int32),
        pltpu.VMEM((3, EMB_DIM), jnp.float32),
        pltpu.VMEM((_BPW, EMB_DIM), jnp.float32),
        pltpu.SemaphoreType.DMA,
        pltpu.SemaphoreType.DMA,
        pltpu.SemaphoreType.DMA,
        pltpu.SemaphoreType.DMA,
        pltpu.SemaphoreType.DMA,
        pltpu.SemaphoreType.DMA,
    ],
)
def _lookup(idx_hbm, table_hbm, out_hbm, idx_v, table_v, rows_v,
            i0, i1, i2, i3, gsem, osem):
    isems = [i0, i1, i2, i3]
    wid = lax.axis_index("s") * _NC + lax.axis_index("c")
    base = wid * _BPW

    idx_copies = [
        pltpu.async_copy(
            idx_hbm.at[pl.ds(base + buf * _ROWS_PER_BUF, _ROWS_PER_BUF)],
            idx_v.at[pl.ds(buf * _ROWS_PER_BUF, _ROWS_PER_BUF)],
            isems[buf])
        for buf in range(_NBUF)
    ]
    pltpu.sync_copy(table_hbm.at[pl.ds(0, 3)], table_v)

    lane3 = lax.iota(jnp.int32, _LANES) * 3
    idx_copies[_GATHER_BUF].wait()
    for g in range(_GPB):
        b0 = _GATHER_BUF * _ROWS_PER_BUF + g * _LANES
        slot0 = ((base + b0) % _REP) * 3
        idx_v[pl.ds(b0, _LANES)] = idx_v[pl.ds(b0, _LANES)] + (slot0 + lane3)
    gsl = pl.ds(_GATHER_BUF * _ROWS_PER_BUF, _ROWS_PER_BUF)
    gather_copy = pltpu.async_copy(
        table_hbm.at[idx_v.at[gsl]], rows_v.at[gsl], gsem)

    t = [[table_v[k, pl.ds(j * _LANES, _LANES)] for j in range(_CHUNKS)]
         for k in range(3)]
    d0 = [t[0][j] - t[2][j] for j in range(_CHUNKS)]
    d1 = [t[1][j] - t[2][j] for j in range(_CHUNKS)]
    one = jnp.ones((_LANES,), jnp.int32)

    out_copies = []
    for buf in range(_NBUF - 1):
        idx_copies[buf].wait()

        def group_body(g, carry):
            b0 = g * _LANES
            idx16 = idx_v[pl.ds(b0, _LANES)]
            for r in range(_LANES):
                bc = _bcast_lane(idx16, r)
                o = jnp.minimum(bc, one)
                w0 = (one - o).astype(jnp.float32)
                w1 = (o - jnp.maximum(bc - one, 0)).astype(jnp.float32)
                for j in range(_CHUNKS):
                    v = t[2][j] + w0 * d0[j] + w1 * d1[j]
                    rows_v[b0 + r, pl.ds(j * _LANES, _LANES)] = v
            return carry

        lax.fori_loop(buf * _GPB, (buf + 1) * _GPB, group_body, 0)
        rsl = pl.ds(buf * _ROWS_PER_BUF, _ROWS_PER_BUF)
        out_copies.append(pltpu.async_copy(
            rows_v.at[rsl],
            out_hbm.at[pl.ds(base + buf * _ROWS_PER_BUF, _ROWS_PER_BUF)],
            osem))

    gather_copy.wait()
    out_copies.append(pltpu.async_copy(
        rows_v.at[gsl],
        out_hbm.at[pl.ds(base + _GATHER_BUF * _ROWS_PER_BUF, _ROWS_PER_BUF)],
        osem))
    for cp in out_copies:
        cp.wait()


def kernel(x, table):
    idx = jnp.reshape(x, (BATCH,)).astype(jnp.int32)
    table_rep = jnp.tile(table, (_REP, 1))
    return _lookup(idx, table_rep)
